# Initial kernel scaffold; baseline (speedup 1.0000x reference)
import jax
import jax.numpy as jnp
from jax.experimental import pallas as pl


def kernel(input, another_input, table1, table2):
    raise NotImplementedError("write your pallas kernel here")


# speedup vs baseline: 1.2484x; 1.2484x over previous
import functools

import jax
import jax.numpy as jnp
from jax import lax
from jax.experimental import pallas as pl
from jax.experimental.pallas import tpu as pltpu
from jax.experimental.pallas import tpu_sc as plsc

_C = 128


@functools.lru_cache(maxsize=None)
def _build(N, D):
    info = plsc.get_sparse_core_info()
    nw = info.num_cores * info.num_subcores
    nper = N // nw
    steps = nper // _C
    mesh = plsc.VectorSubcoreMesh(core_axis_name="c", subcore_axis_name="s")

    def body(t1, i1, t2, i2, out, i1v, i2v, r1, r2, s1, s2):
        wid = lax.axis_index("s") * info.num_cores + lax.axis_index("c")
        base = wid * nper
        pltpu.sync_copy(i1.at[pl.ds(base, nper)], i1v)
        pltpu.sync_copy(i2.at[pl.ds(base, nper)], i2v)

        def step(g, carry):
            o = g * _C
            cp1 = pltpu.async_copy(t1.at[i1v.at[pl.ds(o, _C)]], r1, s1)
            cp2 = pltpu.async_copy(t2.at[i2v.at[pl.ds(o, _C)]], r2, s2)
            cp1.wait()
            cp2.wait()

            def add_row(j, c):
                for h in range(D // 16):
                    sl = pl.ds(h * 16, 16)
                    r1[j, sl] = r1[j, sl] + r2[j, sl]
                return c

            lax.fori_loop(0, _C, add_row, 0, unroll=4)
            pltpu.sync_copy(r1, out.at[pl.ds(base + o, _C)])
            return carry

        lax.fori_loop(0, steps, step, 0)

    return pl.kernel(
        body,
        mesh=mesh,
        out_type=jax.ShapeDtypeStruct((N, D), jnp.float32),
        scratch_types=[
            pltpu.VMEM((nper,), jnp.int32),
            pltpu.VMEM((nper,), jnp.int32),
            pltpu.VMEM((_C, D), jnp.float32),
            pltpu.VMEM((_C, D), jnp.float32),
            pltpu.SemaphoreType.DMA,
            pltpu.SemaphoreType.DMA,
        ],
        compiler_params=pltpu.CompilerParams(use_tc_tiling_on_sc=False),
    )


def kernel(input, another_input, table1, table2):
    B, H = input.shape
    D = table1.shape[1]
    N = B * H
    i1 = input.reshape(N).astype(jnp.int32)
    i2 = another_input.reshape(N).astype(jnp.int32)
    out = _build(N, D)(table1, i1, table2, i2)
    return out.reshape(B, H, D)


# speedup vs baseline: 1.3568x; 1.0868x over previous
import functools

import jax
import jax.numpy as jnp
from jax import lax
from jax.experimental import pallas as pl
from jax.experimental.pallas import tpu as pltpu
from jax.experimental.pallas import tpu_sc as plsc

_C = 128
_NBUF = 4
_OBUF = 2


@functools.lru_cache(maxsize=None)
def _build(N, D):
    info = plsc.get_sparse_core_info()
    nw = info.num_cores * info.num_subcores
    nper = N // nw
    steps = nper // _C
    outer = steps // _NBUF
    mesh = plsc.VectorSubcoreMesh(core_axis_name="c", subcore_axis_name="s")

    def body(t1, i1, t2, i2, out, i1v, i2v, r1, r2, ob, *sems):
        sg1 = sems[:_NBUF]
        sg2 = sems[_NBUF:2 * _NBUF]
        so = sems[2 * _NBUF:]
        wid = lax.axis_index("s") * info.num_cores + lax.axis_index("c")
        base = wid * nper
        pltpu.sync_copy(i1.at[pl.ds(base, nper)], i1v)
        pltpu.sync_copy(i2.at[pl.ds(base, nper)], i2v)

        def issue(g, b):
            o = g * _C
            pltpu.async_copy(t1.at[i1v.at[pl.ds(o, _C)]], r1.at[b], sg1[b])
            pltpu.async_copy(t2.at[i2v.at[pl.ds(o, _C)]], r2.at[b], sg2[b])

        def wait_gather(b):
            pltpu.make_async_copy(
                t1.at[i1v.at[pl.ds(0, _C)]], r1.at[b], sg1[b]).wait()
            pltpu.make_async_copy(
                t2.at[i2v.at[pl.ds(0, _C)]], r2.at[b], sg2[b]).wait()

        def wait_scatter(b2):
            pltpu.make_async_copy(
                ob.at[b2], out.at[pl.ds(0, _C)], so[b2]).wait()

        for b in range(_NBUF):
            issue(b, b)

        def outer_step(g2, carry):
            for b in range(_NBUF):
                g = g2 * _NBUF + b
                wait_gather(b)
                b2 = b % _OBUF
                if b < _OBUF:
                    @pl.when(g2 > 0)
                    def _():
                        wait_scatter(b2)
                else:
                    wait_scatter(b2)

                def add_row(j, c):
                    for h in range(D // 16):
                        sl = pl.ds(h * 16, 16)
                        ob[b2, j, sl] = r1[b, j, sl] + r2[b, j, sl]
                    return c

                lax.fori_loop(0, _C, add_row, 0, unroll=8)

                @pl.when(g2 < outer - 1)
                def _():
                    issue(g + _NBUF, b)

                pltpu.async_copy(ob.at[b2], out.at[pl.ds(base + g * _C, _C)],
                                 so[b2])
            return carry

        lax.fori_loop(0, outer, outer_step, 0)
        for b2 in range(_OBUF):
            wait_scatter(b2)

    return pl.kernel(
        body,
        mesh=mesh,
        out_type=jax.ShapeDtypeStruct((N, D), jnp.float32),
        scratch_types=[
            pltpu.VMEM((nper,), jnp.int32),
            pltpu.VMEM((nper,), jnp.int32),
            pltpu.VMEM((_NBUF, _C, D), jnp.float32),
            pltpu.VMEM((_NBUF, _C, D), jnp.float32),
            pltpu.VMEM((_OBUF, _C, D), jnp.float32),
        ] + [pltpu.SemaphoreType.DMA] * (2 * _NBUF + _OBUF),
        compiler_params=pltpu.CompilerParams(use_tc_tiling_on_sc=False),
    )


def kernel(input, another_input, table1, table2):
    B, H = input.shape
    D = table1.shape[1]
    N = B * H
    i1 = input.reshape(N).astype(jnp.int32)
    i2 = another_input.reshape(N).astype(jnp.int32)
    out = _build(N, D)(table1, i1, table2, i2)
    return out.reshape(B, H, D)


# speedup vs baseline: 2.0587x; 1.5173x over previous
import functools

import jax
import jax.numpy as jnp
from jax import lax
from jax.experimental import pallas as pl
from jax.experimental.pallas import tpu as pltpu
from jax.experimental.pallas import tpu_sc as plsc

_C = 128
_NBUF = 4
_OBUF = 2


@functools.lru_cache(maxsize=None)
def _build(N, D, HB):
    info = plsc.get_sparse_core_info()
    nw = info.num_cores * info.num_subcores
    nper = N // nw
    steps = nper // _C
    outer = steps // _NBUF
    nfa = D // 8
    bblocks = HB // _C
    mesh = plsc.VectorSubcoreMesh(core_axis_name="c", subcore_axis_name="s")

    def body(t1, i1, t2, i2, out, i1v, i2v, r1, r2, ob, *sems):
        sg1 = sems[:_NBUF]
        sg2 = sems[_NBUF:2 * _NBUF]
        so = sems[2 * _NBUF:]
        wid = lax.axis_index("s") * info.num_cores + lax.axis_index("c")
        base = wid * nper
        iota = jax.lax.iota(jnp.int32, 16)
        dconst = []
        for f0 in (0, 16):
            f = f0 + iota
            dconst.append((f >> 3) * (8 * _C) + (f & 7) * _C)
        pltpu.sync_copy(i1.at[pl.ds(base, nper)], i1v)
        pltpu.sync_copy(i2.at[pl.ds(base, nper)], i2v)

        def issue(g, b):
            o = g * _C
            pltpu.async_copy(t1.at[i1v.at[pl.ds(o, _C)]], r1.at[b], sg1[b])
            pltpu.async_copy(t2.at[i2v.at[pl.ds(o, _C)]], r2.at[b], sg2[b])

        def wait_gather(b):
            pltpu.make_async_copy(
                t1.at[i1v.at[pl.ds(0, _C)]], r1.at[b], sg1[b]).wait()
            pltpu.make_async_copy(
                t2.at[i2v.at[pl.ds(0, _C)]], r2.at[b], sg2[b]).wait()

        def wait_scatter(b2):
            for fa in range(nfa):
                pltpu.make_async_copy(
                    ob.at[b2, pl.ds(0, 8 * _C)], out.at[pl.ds(0, 8 * _C)],
                    so[b2]).wait()

        for b in range(_NBUF):
            issue(b, b)

        def outer_step(g2, carry):
            for b in range(_NBUF):
                g = g2 * _NBUF + b
                s = wid * steps + g
                h = s // bblocks
                ba = s % bblocks
                wait_gather(b)
                b2 = b % _OBUF
                if b < _OBUF:
                    @pl.when(g2 > 0)
                    def _():
                        wait_scatter(b2)
                else:
                    wait_scatter(b2)

                def tr_body(bb, c):
                    for f0 in (0, 16):
                        sl = pl.ds(f0, 16)
                        v = r1[b, bb, sl] + r2[b, bb, sl]
                        plsc.store_scatter(ob.at[b2], [dconst[f0 // 16] + bb], v)
                    return c

                lax.fori_loop(0, _C, tr_body, 0, unroll=8)

                @pl.when(g2 < outer - 1)
                def _():
                    issue(g + _NBUF, b)

                for fa in range(nfa):
                    off = (h * nfa + fa) * (bblocks * 8 * _C) + ba * (8 * _C)
                    pltpu.async_copy(ob.at[b2, pl.ds(fa * 8 * _C, 8 * _C)],
                                     out.at[pl.ds(off, 8 * _C)], so[b2])
            return carry

        lax.fori_loop(0, outer, outer_step, 0)
        for b2 in range(_OBUF):
            wait_scatter(b2)

    return pl.kernel(
        body,
        mesh=mesh,
        out_type=jax.ShapeDtypeStruct((N * D,), jnp.float32),
        scratch_types=[
            pltpu.VMEM((nper,), jnp.int32),
            pltpu.VMEM((nper,), jnp.int32),
            pltpu.VMEM((_NBUF, _C, D), jnp.float32),
            pltpu.VMEM((_NBUF, _C, D), jnp.float32),
            pltpu.VMEM((_OBUF, nfa * 8 * _C), jnp.float32),
        ] + [pltpu.SemaphoreType.DMA] * (2 * _NBUF + _OBUF),
        compiler_params=pltpu.CompilerParams(use_tc_tiling_on_sc=False,
                                             needs_layout_passes=False),
    )


def kernel(input, another_input, table1, table2):
    B, H = input.shape
    D = table1.shape[1]
    N = B * H
    i1 = input.T.reshape(N).astype(jnp.int32)
    i2 = another_input.T.reshape(N).astype(jnp.int32)
    flat = _build(N, D, B)(table1, i1, table2, i2)
    out5 = flat.reshape(H, D // 8, B // 128, 8, 128)
    return out5.transpose(2, 4, 0, 1, 3).reshape(B, H, D)
